# baseline (speedup 1.0000x reference)
import functools

import jax
import jax.numpy as jnp
from jax import lax
from jax.experimental import pallas as pl
from jax.experimental.pallas import tpu as pltpu


T = 256


def _mm_body(te_ref, x_ref, w_ref, b_ref, y_ref):
    x = x_ref[...]
    w = w_ref[0]
    y = lax.dot_general(x, w, (((1,), (1,)), ((), ())),
                        preferred_element_type=jnp.float32)
    y_ref[...] = y + b_ref[0]


def _grouped_matmul(tile_expert, x_padded, W, b):
    P, D = x_padded.shape
    nt = P // T
    grid_spec = pltpu.PrefetchScalarGridSpec(
        num_scalar_prefetch=1,
        grid=(nt,),
        in_specs=[
            pl.BlockSpec((T, D), lambda i, te: (i, 0)),
            pl.BlockSpec((1, D, D), lambda i, te: (te[i], 0, 0)),
            pl.BlockSpec((1, 1, D), lambda i, te: (te[i], 0, 0)),
        ],
        out_specs=pl.BlockSpec((T, D), lambda i, te: (i, 0)),
    )
    return pl.pallas_call(
        _mm_body,
        grid_spec=grid_spec,
        out_shape=jax.ShapeDtypeStruct((P, D), jnp.float32),
    )(tile_expert, x_padded, W, b.reshape(b.shape[0], 1, b.shape[1]))


def kernel(xs, mxs, actions, W, b):
    N, D = xs.shape
    E = W.shape[0]
    a = actions.astype(jnp.int32)

    oh = (a[:, None] == jnp.arange(E, dtype=jnp.int32)[None, :]).astype(jnp.int32)
    ranks = jnp.cumsum(oh, axis=0) - 1
    rank = jnp.take_along_axis(ranks, a[:, None], axis=1)[:, 0]
    hist = ranks[-1] + 1
    padded = ((hist + T - 1) // T) * T
    off_end = jnp.cumsum(padded)
    off = off_end - padded
    P = N + E * T
    p = off[a] + rank
    src = jnp.zeros((P,), jnp.int32).at[p].set(
        jnp.arange(N, dtype=jnp.int32))
    tile_start = jnp.arange(P // T, dtype=jnp.int32) * T
    tile_expert = jnp.minimum(
        jnp.searchsorted(off_end, tile_start, side="right").astype(jnp.int32),
        E - 1)

    x_padded = xs[src]

    y_padded = _grouped_matmul(tile_expert, x_padded, W, b)

    ys = y_padded[p]
    return (ys, mxs, actions)


# speedup vs baseline: 1.0088x; 1.0088x over previous
import functools

import jax
import jax.numpy as jnp
from jax import lax
from jax.experimental import pallas as pl
from jax.experimental.pallas import tpu as pltpu
from jax.experimental.pallas import tpu_sc as plsc


T = 256
_CH = 32


@functools.lru_cache(maxsize=None)
def _make_sc_row_gather(R, D, B):
    info = plsc.get_sparse_core_info()
    nw = info.num_cores * info.num_subcores
    b_per_w = B // nw
    assert B % (8 * nw) == 0 and b_per_w % _CH == 0
    n_ch = b_per_w // _CH
    nc = info.num_cores
    mesh = plsc.VectorSubcoreMesh(core_axis_name="c", subcore_axis_name="s")

    @functools.partial(
        pl.kernel,
        out_type=jax.ShapeDtypeStruct((B, D), jnp.float32),
        mesh=mesh,
        scratch_types=[
            pltpu.VMEM((b_per_w,), jnp.int32),
            pltpu.VMEM((_CH, D), jnp.float32),
            pltpu.VMEM((_CH, D), jnp.float32),
            pltpu.SemaphoreType.DMA,
            pltpu.SemaphoreType.DMA,
            pltpu.SemaphoreType.DMA,
        ],
    )
    def k(table_hbm, idx_hbm, out_hbm, idx_v, buf0, buf1, gsem, ssem0, ssem1):
        wid = lax.axis_index("s") * nc + lax.axis_index("c")
        base = wid * b_per_w
        pltpu.sync_copy(idx_hbm.at[pl.ds(base, b_per_w)], idx_v)
        bufs = (buf0, buf1)
        ssems = (ssem0, ssem1)
        for c in range(n_ch):
            buf = bufs[c % 2]
            ssem = ssems[c % 2]
            if c >= 2:
                pltpu.make_async_copy(
                    buf, out_hbm.at[pl.ds(base + (c - 2) * _CH, _CH)], ssem
                ).wait()
            pltpu.async_copy(
                table_hbm.at[idx_v.at[pl.ds(c * _CH, _CH)]], buf, gsem
            ).wait()
            pltpu.async_copy(buf, out_hbm.at[pl.ds(base + c * _CH, _CH)], ssem)
        for c in range(max(n_ch - 2, 0), n_ch):
            pltpu.make_async_copy(
                bufs[c % 2], out_hbm.at[pl.ds(base + c * _CH, _CH)], ssems[c % 2]
            ).wait()

    return k


def _mm_body(te_ref, x_ref, w_ref, b_ref, y_ref):
    x = x_ref[...]
    w = w_ref[0]
    y = lax.dot_general(x, w, (((1,), (1,)), ((), ())),
                        preferred_element_type=jnp.float32)
    y_ref[...] = y + b_ref[0]


def _grouped_matmul(tile_expert, x_padded, W, b):
    P, D = x_padded.shape
    nt = P // T
    grid_spec = pltpu.PrefetchScalarGridSpec(
        num_scalar_prefetch=1,
        grid=(nt,),
        in_specs=[
            pl.BlockSpec((T, D), lambda i, te: (i, 0)),
            pl.BlockSpec((1, D, D), lambda i, te: (te[i], 0, 0)),
            pl.BlockSpec((1, 1, D), lambda i, te: (te[i], 0, 0)),
        ],
        out_specs=pl.BlockSpec((T, D), lambda i, te: (i, 0)),
    )
    return pl.pallas_call(
        _mm_body,
        grid_spec=grid_spec,
        out_shape=jax.ShapeDtypeStruct((P, D), jnp.float32),
    )(tile_expert, x_padded, W, b.reshape(b.shape[0], 1, b.shape[1]))


def kernel(xs, mxs, actions, W, b):
    N, D = xs.shape
    E = W.shape[0]
    a = actions.astype(jnp.int32)

    oh = (a[:, None] == jnp.arange(E, dtype=jnp.int32)[None, :]).astype(jnp.int32)
    ranks = jnp.cumsum(oh, axis=0) - 1
    rank = jnp.take_along_axis(ranks, a[:, None], axis=1)[:, 0]
    hist = ranks[-1] + 1
    padded = ((hist + T - 1) // T) * T
    off_end = jnp.cumsum(padded)
    off = off_end - padded
    P = N + E * T
    p = off[a] + rank
    src = jnp.zeros((P,), jnp.int32).at[p].set(
        jnp.arange(N, dtype=jnp.int32))
    tile_start = jnp.arange(P // T, dtype=jnp.int32) * T
    tile_expert = jnp.minimum(
        jnp.searchsorted(off_end, tile_start, side="right").astype(jnp.int32),
        E - 1)

    x_padded = _make_sc_row_gather(N, D, P)(xs, src)

    y_padded = _grouped_matmul(tile_expert, x_padded, W, b)

    ys = _make_sc_row_gather(P, D, N)(y_padded, p)
    return (ys, mxs, actions)


# speedup vs baseline: 2.1436x; 2.1250x over previous
import functools

import jax
import jax.numpy as jnp
from jax import lax
from jax.experimental import pallas as pl
from jax.experimental.pallas import tpu as pltpu
from jax.experimental.pallas import tpu_sc as plsc


T = 256
_CH = 32


@functools.lru_cache(maxsize=None)
def _make_sc_row_gather(R, D, B):
    info = plsc.get_sparse_core_info()
    nw = info.num_cores * info.num_subcores
    b_per_w = B // nw
    assert B % (8 * nw) == 0 and b_per_w % _CH == 0
    n_ch = b_per_w // _CH
    nc = info.num_cores
    mesh = plsc.VectorSubcoreMesh(core_axis_name="c", subcore_axis_name="s")

    @functools.partial(
        pl.kernel,
        out_type=jax.ShapeDtypeStruct((B, D), jnp.float32),
        mesh=mesh,
        scratch_types=[
            pltpu.VMEM((b_per_w,), jnp.int32),
            pltpu.VMEM((_CH, D), jnp.float32),
            pltpu.VMEM((_CH, D), jnp.float32),
            pltpu.SemaphoreType.DMA,
            pltpu.SemaphoreType.DMA,
            pltpu.SemaphoreType.DMA,
        ],
    )
    def k(table_hbm, idx_hbm, out_hbm, idx_v, buf0, buf1, gsem, ssem0, ssem1):
        wid = lax.axis_index("s") * nc + lax.axis_index("c")
        base = wid * b_per_w
        pltpu.sync_copy(idx_hbm.at[pl.ds(base, b_per_w)], idx_v)
        bufs = (buf0, buf1)
        ssems = (ssem0, ssem1)
        for c in range(n_ch):
            buf = bufs[c % 2]
            ssem = ssems[c % 2]
            if c >= 2:
                pltpu.make_async_copy(
                    buf, out_hbm.at[pl.ds(base + (c - 2) * _CH, _CH)], ssem
                ).wait()
            pltpu.async_copy(
                table_hbm.at[idx_v.at[pl.ds(c * _CH, _CH)]], buf, gsem
            ).wait()
            pltpu.async_copy(buf, out_hbm.at[pl.ds(base + c * _CH, _CH)], ssem)
        for c in range(max(n_ch - 2, 0), n_ch):
            pltpu.make_async_copy(
                bufs[c % 2], out_hbm.at[pl.ds(base + c * _CH, _CH)], ssems[c % 2]
            ).wait()

    return k


@functools.lru_cache(maxsize=None)
def _make_sc_row_scatter(B, D, R):
    info = plsc.get_sparse_core_info()
    nw = info.num_cores * info.num_subcores
    b_per_w = B // nw
    assert B % (8 * nw) == 0 and b_per_w % _CH == 0
    n_ch = b_per_w // _CH
    nc = info.num_cores
    mesh = plsc.VectorSubcoreMesh(core_axis_name="c", subcore_axis_name="s")

    @functools.partial(
        pl.kernel,
        out_type=jax.ShapeDtypeStruct((R, D), jnp.float32),
        mesh=mesh,
        scratch_types=[
            pltpu.VMEM((n_ch, _CH), jnp.int32),
            pltpu.VMEM((_CH, D), jnp.float32),
            pltpu.VMEM((_CH, D), jnp.float32),
            pltpu.SemaphoreType.DMA,
            pltpu.SemaphoreType.DMA,
        ],
    )
    def k(table_hbm, idx_hbm, out_hbm, idx_v, buf0, buf1, ssem0, ssem1):
        wid = lax.axis_index("s") * nc + lax.axis_index("c")
        base = wid * b_per_w
        pltpu.sync_copy(idx_hbm.at[wid], idx_v)
        bufs = (buf0, buf1)
        ssems = (ssem0, ssem1)
        for c in range(n_ch):
            buf = bufs[c % 2]
            ssem = ssems[c % 2]
            if c >= 2:
                pltpu.make_async_copy(
                    buf, out_hbm.at[idx_v.at[c - 2]], ssem
                ).wait()
            pltpu.sync_copy(table_hbm.at[pl.ds(base + c * _CH, _CH)], buf)
            pltpu.async_copy(buf, out_hbm.at[idx_v.at[c]], ssem)
        for c in range(max(n_ch - 2, 0), n_ch):
            pltpu.make_async_copy(
                bufs[c % 2], out_hbm.at[idx_v.at[c]], ssems[c % 2]
            ).wait()

    return k


def _mm_body(te_ref, x_ref, w_ref, b_ref, y_ref):
    x = x_ref[...]
    w = w_ref[0]
    y = lax.dot_general(x, w, (((1,), (1,)), ((), ())),
                        preferred_element_type=jnp.float32)
    y_ref[...] = y + b_ref[0]


def _grouped_matmul(tile_expert, x_padded, W, b):
    P, D = x_padded.shape
    nt = P // T
    grid_spec = pltpu.PrefetchScalarGridSpec(
        num_scalar_prefetch=1,
        grid=(nt,),
        in_specs=[
            pl.BlockSpec((T, D), lambda i, te: (i, 0)),
            pl.BlockSpec((1, D, D), lambda i, te: (te[i], 0, 0)),
            pl.BlockSpec((1, 1, D), lambda i, te: (te[i], 0, 0)),
        ],
        out_specs=pl.BlockSpec((T, D), lambda i, te: (i, 0)),
    )
    return pl.pallas_call(
        _mm_body,
        grid_spec=grid_spec,
        out_shape=jax.ShapeDtypeStruct((P, D), jnp.float32),
    )(tile_expert, x_padded, W, b.reshape(b.shape[0], 1, b.shape[1]))


def kernel(xs, mxs, actions, W, b):
    N, D = xs.shape
    E = W.shape[0]
    a = actions.astype(jnp.int32)

    oh = (a[:, None] == jnp.arange(E, dtype=jnp.int32)[None, :]).astype(jnp.int32)
    ranks = jnp.cumsum(oh, axis=0) - 1
    rank = jnp.take_along_axis(ranks, a[:, None], axis=1)[:, 0]
    hist = ranks[-1] + 1
    padded = ((hist + T - 1) // T) * T
    off_end = jnp.cumsum(padded)
    off = off_end - padded
    P = N + E * T
    p = off[a] + rank
    tile_start = jnp.arange(P // T, dtype=jnp.int32) * T
    tile_expert = jnp.minimum(
        jnp.searchsorted(off_end, tile_start, side="right").astype(jnp.int32),
        E - 1)

    nw = 32
    p3 = p.reshape(nw, -1, _CH)
    x_padded = _make_sc_row_scatter(N, D, P)(xs, p3)

    y_padded = _grouped_matmul(tile_expert, x_padded, W, b)

    ys = _make_sc_row_gather(P, D, N)(y_padded, p)
    return (ys, mxs, actions)
